# baseline (speedup 1.0000x reference)
import functools

import jax
import jax.numpy as jnp
import numpy as np
from jax import lax
from jax.experimental import pallas as pl
from jax.experimental.pallas import tpu as pltpu
from jax.experimental.pallas import tpu_sc as plsc

_MAX_SEQ_LEN = 4096


def _position_enc_table(max_seq_len: int, dim: int) -> jnp.ndarray:
    pos = np.arange(max_seq_len, dtype=np.float64)[:, None]
    i = np.arange(dim, dtype=np.float64)[None, :]
    enc = pos / np.power(10000.0, (i - (i % 2)) / dim)
    enc[:, 0::2] = np.sin(enc[:, 0::2])
    enc[:, 1::2] = np.cos(enc[:, 1::2])
    return jnp.asarray(enc, dtype=jnp.float32)


def _broadcast_rows(table, b, t, dim):
    info = plsc.get_sparse_core_info()
    nw = info.num_cores * info.num_subcores
    rows_per_w = t // nw
    chunk = min(rows_per_w, 32)
    n_chunks = rows_per_w // chunk
    mesh = plsc.VectorSubcoreMesh(core_axis_name="c", subcore_axis_name="s")

    @functools.partial(
        pl.kernel,
        mesh=mesh,
        out_type=jax.ShapeDtypeStruct((b * t, dim), jnp.float32),
        scratch_types=[
            pltpu.VMEM((2, chunk, dim), jnp.float32),
            pltpu.SemaphoreType.DMA,
            pltpu.SemaphoreType.DMA,
            pltpu.SemaphoreType.DMA,
        ],
    )
    def k(table_hbm, out_hbm, buf, ld_sem, st_sem0, st_sem1):
        wid = lax.axis_index("s") * info.num_cores + lax.axis_index("c")
        base = wid * rows_per_w
        st_sems = (st_sem0, st_sem1)

        def start_load(c):
            return pltpu.async_copy(
                table_hbm.at[pl.ds(base + c * chunk, chunk)], buf.at[c % 2], ld_sem
            )

        loads = [None] * n_chunks
        stores = [[] for _ in range(n_chunks)]
        loads[0] = start_load(0)
        for c in range(n_chunks):
            loads[c].wait()
            if c + 1 < n_chunks:
                if c >= 1:
                    for d in stores[c - 1]:
                        d.wait()
                loads[c + 1] = start_load(c + 1)
            row0 = base + c * chunk
            for bb in range(b):
                stores[c].append(
                    pltpu.async_copy(
                        buf.at[c % 2],
                        out_hbm.at[pl.ds(bb * t + row0, chunk)],
                        st_sems[c % 2],
                    )
                )
        for c in (n_chunks - 2, n_chunks - 1):
            if c >= 0 and stores[c]:
                for d in stores[c]:
                    d.wait()
                stores[c] = []

    return k(table).reshape(b, t, dim)


def kernel(inputs):
    b, t, dim = inputs.shape
    table = _position_enc_table(_MAX_SEQ_LEN, dim)[:t]
    return _broadcast_rows(table, b, t, dim)
